# Initial kernel scaffold; baseline (speedup 1.0000x reference)
import jax
import jax.numpy as jnp
from jax.experimental import pallas as pl


def kernel(x, table, W_ih, W_hh, b_ih, b_hh):
    raise NotImplementedError("write your pallas kernel here")


# speedup vs baseline: 3.0237x; 3.0237x over previous
import functools

import jax
import jax.numpy as jnp
from jax import lax
from jax.experimental import pallas as pl
from jax.experimental.pallas import tpu as pltpu
from jax.experimental.pallas import tpu_sc as plsc

EMB = 32
HID = 64
BATCH = 4096
SEQ = 50

_NW = 32
_ROWS = BATCH * SEQ
_RPW = _ROWS // _NW
_GC = 128
_GPW = _RPW // _GC
_CH = 10
_NCH = _GPW // _CH
_CROWS = _CH * _GC


def _gather_sc(idx3d, table):
    mesh = plsc.VectorSubcoreMesh(core_axis_name="c", subcore_axis_name="s")

    @functools.partial(
        pl.kernel,
        mesh=mesh,
        out_type=jax.ShapeDtypeStruct((_ROWS, EMB), jnp.float32),
        scratch_types=[
            pltpu.VMEM((_CH, _GC), jnp.int32),
            pltpu.VMEM((_CROWS, EMB), jnp.float32),
            pltpu.SemaphoreType.DMA,
        ],
        compiler_params=pltpu.CompilerParams(use_tc_tiling_on_sc=False),
    )
    def k(idx_hbm, table_hbm, out_hbm, idx_v, rows_v, sem):
        wid = lax.axis_index("s") * 2 + lax.axis_index("c")

        def chunk(c, carry):
            blk = wid * _NCH + c
            pltpu.sync_copy(idx_hbm.at[blk], idx_v)
            for j in range(_CH):
                pltpu.async_copy(
                    table_hbm.at[idx_v.at[j]],
                    rows_v.at[pl.ds(j * _GC, _GC)],
                    sem,
                )
            for j in range(_CH):
                pltpu.make_async_copy(
                    table_hbm.at[idx_v.at[j]],
                    rows_v.at[pl.ds(j * _GC, _GC)],
                    sem,
                ).wait()
            pltpu.sync_copy(rows_v, out_hbm.at[pl.ds(blk * _CROWS, _CROWS)])
            return carry

        lax.fori_loop(0, _NCH, chunk, 0)

    return k(idx3d, table)


def _lstm_tc(emb_t, a_mat, b_mat, bias):
    bb = 512

    def body(emb_ref, a_ref, b_ref, bias_ref, out_ref):
        a = a_ref[...]
        bm = b_ref[...]
        bias_v = bias_ref[...]

        def step(t, carry):
            h, c = carry
            x_t = emb_ref[t]
            gates = (
                jnp.dot(x_t, a, preferred_element_type=jnp.float32)
                + jnp.dot(h, bm, preferred_element_type=jnp.float32)
                + bias_v
            )
            i = jax.nn.sigmoid(gates[:, 0:HID])
            f = jax.nn.sigmoid(gates[:, HID:2 * HID])
            g = jnp.tanh(gates[:, 2 * HID:3 * HID])
            o = jax.nn.sigmoid(gates[:, 3 * HID:4 * HID])
            c2 = f * c + i * g
            h2 = o * jnp.tanh(c2)
            return (h2, c2)

        h0 = jnp.zeros((bb, HID), jnp.float32)
        c0 = jnp.zeros((bb, HID), jnp.float32)
        h, _ = lax.fori_loop(0, SEQ, step, (h0, c0))
        out_ref[...] = h

    return pl.pallas_call(
        body,
        grid=(BATCH // bb,),
        in_specs=[
            pl.BlockSpec((SEQ, bb, EMB), lambda i: (0, i, 0)),
            pl.BlockSpec((EMB, 4 * HID), lambda i: (0, 0)),
            pl.BlockSpec((HID, 4 * HID), lambda i: (0, 0)),
            pl.BlockSpec((1, 4 * HID), lambda i: (0, 0)),
        ],
        out_specs=pl.BlockSpec((bb, HID), lambda i: (i, 0)),
        out_shape=jax.ShapeDtypeStruct((BATCH, HID), jnp.float32),
    )(emb_t, a_mat, b_mat, bias)


def kernel(x, table, W_ih, W_hh, b_ih, b_hh):
    idx3d = x.T.reshape(_NW * _NCH, _CH, _GC)
    emb = _gather_sc(idx3d, table)
    emb_t = emb.reshape(SEQ, BATCH, EMB)
    a_mat = W_ih.T
    b_mat = W_hh.T
    bias = (b_ih + b_hh).reshape(1, 4 * HID)
    return _lstm_tc(emb_t, a_mat, b_mat, bias)


# speedup vs baseline: 3.6923x; 1.2211x over previous
import functools

import jax
import jax.numpy as jnp
from jax import lax
from jax.experimental import pallas as pl
from jax.experimental.pallas import tpu as pltpu
from jax.experimental.pallas import tpu_sc as plsc

EMB = 32
HID = 64
BATCH = 4096
SEQ = 50

_NW = 32
_ROWS = BATCH * SEQ
_RPW = _ROWS // _NW
_GC = 128
_GPW = _RPW // _GC
_CH = 10
_NCH = _GPW // _CH
_CROWS = _CH * _GC


def _gather_sc(idx3d, table):
    mesh = plsc.VectorSubcoreMesh(core_axis_name="c", subcore_axis_name="s")

    @functools.partial(
        pl.kernel,
        mesh=mesh,
        out_type=jax.ShapeDtypeStruct((_ROWS, EMB), jnp.float32),
        scratch_types=[
            pltpu.VMEM((_CH, _GC), jnp.int32),
            pltpu.VMEM((_CROWS, EMB), jnp.float32),
            pltpu.SemaphoreType.DMA,
        ],
        compiler_params=pltpu.CompilerParams(use_tc_tiling_on_sc=False),
    )
    def k(idx_hbm, table_hbm, out_hbm, idx_v, rows_v, sem):
        wid = lax.axis_index("s") * 2 + lax.axis_index("c")

        def chunk(c, carry):
            blk = wid * _NCH + c
            pltpu.sync_copy(idx_hbm.at[blk], idx_v)
            for j in range(_CH):
                pltpu.async_copy(
                    table_hbm.at[idx_v.at[j]],
                    rows_v.at[pl.ds(j * _GC, _GC)],
                    sem,
                )
            for j in range(_CH):
                pltpu.make_async_copy(
                    table_hbm.at[idx_v.at[j]],
                    rows_v.at[pl.ds(j * _GC, _GC)],
                    sem,
                ).wait()
            pltpu.sync_copy(rows_v, out_hbm.at[pl.ds(blk * _CROWS, _CROWS)])
            return carry

        lax.fori_loop(0, _NCH, chunk, 0)

    return k(idx3d, table)


def _lstm_tc(emb_t, a_mat, b_mat, bias):
    def body(emb_ref, a_ref, b_ref, bias_ref, out_ref, h_ref, c_ref):
        t = pl.program_id(0)

        @pl.when(t == 0)
        def _init():
            h_ref[...] = jnp.zeros((BATCH, HID), jnp.float32)
            c_ref[...] = jnp.zeros((BATCH, HID), jnp.float32)

        x_t = emb_ref[0]
        h = h_ref[...]
        c = c_ref[...]
        gates = (
            jnp.dot(x_t, a_ref[...], preferred_element_type=jnp.float32)
            + jnp.dot(h, b_ref[...], preferred_element_type=jnp.float32)
            + bias_ref[...]
        )
        i = jax.nn.sigmoid(gates[:, 0:HID])
        f = jax.nn.sigmoid(gates[:, HID:2 * HID])
        g = jnp.tanh(gates[:, 2 * HID:3 * HID])
        o = jax.nn.sigmoid(gates[:, 3 * HID:4 * HID])
        c2 = f * c + i * g
        h2 = o * jnp.tanh(c2)
        h_ref[...] = h2
        c_ref[...] = c2

        @pl.when(t == SEQ - 1)
        def _fin():
            out_ref[...] = h2

    return pl.pallas_call(
        body,
        grid=(SEQ,),
        in_specs=[
            pl.BlockSpec((1, BATCH, EMB), lambda i: (i, 0, 0)),
            pl.BlockSpec((EMB, 4 * HID), lambda i: (0, 0)),
            pl.BlockSpec((HID, 4 * HID), lambda i: (0, 0)),
            pl.BlockSpec((1, 4 * HID), lambda i: (0, 0)),
        ],
        out_specs=pl.BlockSpec((BATCH, HID), lambda i: (0, 0)),
        out_shape=jax.ShapeDtypeStruct((BATCH, HID), jnp.float32),
        scratch_shapes=[
            pltpu.VMEM((BATCH, HID), jnp.float32),
            pltpu.VMEM((BATCH, HID), jnp.float32),
        ],
    )(emb_t, a_mat, b_mat, bias)


def kernel(x, table, W_ih, W_hh, b_ih, b_hh):
    idx3d = x.T.reshape(_NW * _NCH, _CH, _GC)
    emb = _gather_sc(idx3d, table)
    emb_t = emb.reshape(SEQ, BATCH, EMB)
    a_mat = W_ih.T
    b_mat = W_hh.T
    bias = (b_ih + b_hh).reshape(1, 4 * HID)
    return _lstm_tc(emb_t, a_mat, b_mat, bias)


# speedup vs baseline: 4.2688x; 1.1561x over previous
import functools

import jax
import jax.numpy as jnp
from jax import lax
from jax.experimental import pallas as pl
from jax.experimental.pallas import tpu as pltpu
from jax.experimental.pallas import tpu_sc as plsc

EMB = 32
HID = 64
BATCH = 4096
SEQ = 50

_NW = 32
_ROWS = BATCH * SEQ
_RPW = _ROWS // _NW
_GC = 128
_GPW = _RPW // _GC
_CH = 10
_NCH = _GPW // _CH
_CROWS = _CH * _GC


def _gather_sc(idx3d, table):
    mesh = plsc.VectorSubcoreMesh(core_axis_name="c", subcore_axis_name="s")

    @functools.partial(
        pl.kernel,
        mesh=mesh,
        out_type=jax.ShapeDtypeStruct((_ROWS, EMB), jnp.float32),
        scratch_types=[
            pltpu.VMEM((_CH, _GC), jnp.int32),
            pltpu.VMEM((_CROWS, EMB), jnp.float32),
            pltpu.SemaphoreType.DMA,
        ],
        compiler_params=pltpu.CompilerParams(use_tc_tiling_on_sc=False),
    )
    def k(idx_hbm, table_hbm, out_hbm, idx_v, rows_v, sem):
        wid = lax.axis_index("s") * 2 + lax.axis_index("c")

        def chunk(c, carry):
            blk = wid * _NCH + c
            pltpu.sync_copy(idx_hbm.at[blk], idx_v)
            for j in range(_CH):
                pltpu.async_copy(
                    table_hbm.at[idx_v.at[j]],
                    rows_v.at[pl.ds(j * _GC, _GC)],
                    sem,
                )
            for j in range(_CH):
                pltpu.make_async_copy(
                    table_hbm.at[idx_v.at[j]],
                    rows_v.at[pl.ds(j * _GC, _GC)],
                    sem,
                ).wait()
            pltpu.sync_copy(rows_v, out_hbm.at[pl.ds(blk * _CROWS, _CROWS)])
            return carry

        lax.fori_loop(0, _NCH, chunk, 0)

    return k(idx3d, table)


def _lstm_tc(emb_t, a_mat, b_mat, bias):
    def body(emb_ref, a_ref, b_ref, bias_ref, out_ref, h_ref, c_ref):
        t = pl.program_id(0)

        @pl.when(t == 0)
        def _init():
            h_ref[...] = jnp.zeros((HID, BATCH), jnp.float32)
            c_ref[...] = jnp.zeros((HID, BATCH), jnp.float32)

        x_t = emb_ref[0].T
        h = h_ref[...]
        c = c_ref[...]
        gates = (
            jnp.dot(a_ref[...], x_t, preferred_element_type=jnp.float32)
            + jnp.dot(b_ref[...], h, preferred_element_type=jnp.float32)
            + bias_ref[...]
        )
        i = jax.nn.sigmoid(gates[0:HID])
        f = jax.nn.sigmoid(gates[HID:2 * HID])
        g = jnp.tanh(gates[2 * HID:3 * HID])
        o = jax.nn.sigmoid(gates[3 * HID:4 * HID])
        c2 = f * c + i * g
        h2 = o * jnp.tanh(c2)
        h_ref[...] = h2
        c_ref[...] = c2

        @pl.when(t == SEQ - 1)
        def _fin():
            out_ref[...] = h2.T

    return pl.pallas_call(
        body,
        grid=(SEQ,),
        in_specs=[
            pl.BlockSpec((1, BATCH, EMB), lambda i: (i, 0, 0)),
            pl.BlockSpec((4 * HID, EMB), lambda i: (0, 0)),
            pl.BlockSpec((4 * HID, HID), lambda i: (0, 0)),
            pl.BlockSpec((4 * HID, 1), lambda i: (0, 0)),
        ],
        out_specs=pl.BlockSpec((BATCH, HID), lambda i: (0, 0)),
        out_shape=jax.ShapeDtypeStruct((BATCH, HID), jnp.float32),
        scratch_shapes=[
            pltpu.VMEM((HID, BATCH), jnp.float32),
            pltpu.VMEM((HID, BATCH), jnp.float32),
        ],
    )(emb_t, a_mat, b_mat, bias)


def kernel(x, table, W_ih, W_hh, b_ih, b_hh):
    idx3d = x.T.reshape(_NW * _NCH, _CH, _GC)
    emb = _gather_sc(idx3d, table)
    emb_t = emb.reshape(SEQ, BATCH, EMB)
    bias = (b_ih + b_hh).reshape(4 * HID, 1)
    return _lstm_tc(emb_t, W_ih, W_hh, bias)


# speedup vs baseline: 4.8060x; 1.1258x over previous
import functools

import jax
import jax.numpy as jnp
from jax import lax
from jax.experimental import pallas as pl
from jax.experimental.pallas import tpu as pltpu
from jax.experimental.pallas import tpu_sc as plsc

EMB = 32
HID = 64
BATCH = 4096
SEQ = 50

_NW = 32
_ROWS = BATCH * SEQ
_RPW = _ROWS // _NW
_GC = 128
_GPW = _RPW // _GC
_CH = 10
_NCH = _GPW // _CH
_CROWS = _CH * _GC


def _gather_sc(idx3d, table):
    mesh = plsc.VectorSubcoreMesh(core_axis_name="c", subcore_axis_name="s")

    @functools.partial(
        pl.kernel,
        mesh=mesh,
        out_type=jax.ShapeDtypeStruct((_ROWS, EMB), jnp.float32),
        scratch_types=[
            pltpu.VMEM((_CH, _GC), jnp.int32),
            pltpu.VMEM((_CROWS, EMB), jnp.float32),
            pltpu.SemaphoreType.DMA,
        ],
        compiler_params=pltpu.CompilerParams(use_tc_tiling_on_sc=False),
    )
    def k(idx_hbm, table_hbm, out_hbm, idx_v, rows_v, sem):
        wid = lax.axis_index("s") * 2 + lax.axis_index("c")

        def chunk(c, carry):
            blk = wid * _NCH + c
            pltpu.sync_copy(idx_hbm.at[blk], idx_v)
            for j in range(_CH):
                pltpu.async_copy(
                    table_hbm.at[idx_v.at[j]],
                    rows_v.at[pl.ds(j * _GC, _GC)],
                    sem,
                )
            for j in range(_CH):
                pltpu.make_async_copy(
                    table_hbm.at[idx_v.at[j]],
                    rows_v.at[pl.ds(j * _GC, _GC)],
                    sem,
                ).wait()
            pltpu.sync_copy(rows_v, out_hbm.at[pl.ds(blk * _CROWS, _CROWS)])
            return carry

        lax.fori_loop(0, _NCH, chunk, 0)

    return k(idx3d, table)


_KDIM = 104


def _sig(v):
    return 0.5 * jnp.tanh(0.5 * v) + 0.5


def _lstm_tc(emb_t, w_comb):
    def body(emb_ref, w_ref, out_ref, xh_ref, c_ref):
        t = pl.program_id(0)

        @pl.when(t == 0)
        def _init():
            row = lax.broadcasted_iota(jnp.int32, (_KDIM - EMB, BATCH), 0)
            xh_ref[EMB:_KDIM] = jnp.where(row == HID, 1.0, 0.0)
            c_ref[...] = jnp.zeros((HID, BATCH), jnp.float32)

        xh_ref[0:EMB] = emb_ref[0].T
        c = c_ref[...]
        gates = jnp.dot(w_ref[...], xh_ref[...],
                        preferred_element_type=jnp.float32)
        i = _sig(gates[0:HID])
        f = _sig(gates[HID:2 * HID])
        g = jnp.tanh(gates[2 * HID:3 * HID])
        o = _sig(gates[3 * HID:4 * HID])
        c2 = f * c + i * g
        h2 = o * jnp.tanh(c2)
        xh_ref[EMB:EMB + HID] = h2
        c_ref[...] = c2

        @pl.when(t == SEQ - 1)
        def _fin():
            out_ref[...] = h2.T

    return pl.pallas_call(
        body,
        grid=(SEQ,),
        in_specs=[
            pl.BlockSpec((1, BATCH, EMB), lambda i: (i, 0, 0)),
            pl.BlockSpec((4 * HID, _KDIM), lambda i: (0, 0)),
        ],
        out_specs=pl.BlockSpec((BATCH, HID), lambda i: (0, 0)),
        out_shape=jax.ShapeDtypeStruct((BATCH, HID), jnp.float32),
        scratch_shapes=[
            pltpu.VMEM((_KDIM, BATCH), jnp.float32),
            pltpu.VMEM((HID, BATCH), jnp.float32),
        ],
    )(emb_t, w_comb)


def kernel(x, table, W_ih, W_hh, b_ih, b_hh):
    idx3d = x.T.reshape(_NW * _NCH, _CH, _GC)
    emb = _gather_sc(idx3d, table)
    emb_t = emb.reshape(SEQ, BATCH, EMB)
    w_comb = jnp.concatenate(
        [W_ih, W_hh, (b_ih + b_hh).reshape(4 * HID, 1),
         jnp.zeros((4 * HID, _KDIM - EMB - HID - 1), jnp.float32)],
        axis=1,
    )
    return _lstm_tc(emb_t, w_comb)


# speedup vs baseline: 5.5368x; 1.1521x over previous
import functools

import jax
import jax.numpy as jnp
from jax import lax
from jax.experimental import pallas as pl
from jax.experimental.pallas import tpu as pltpu
from jax.experimental.pallas import tpu_sc as plsc

EMB = 32
HID = 64
BATCH = 4096
SEQ = 50

_NW = 32
_ROWS = BATCH * SEQ
_RPW = _ROWS // _NW
_GC = 128
_GPW = _RPW // _GC
_CH = 10
_NCH = _GPW // _CH
_CROWS = _CH * _GC


def _gather_sc(idx3d, table):
    mesh = plsc.VectorSubcoreMesh(core_axis_name="c", subcore_axis_name="s")

    @functools.partial(
        pl.kernel,
        mesh=mesh,
        out_type=jax.ShapeDtypeStruct((_ROWS, EMB), jnp.float32),
        scratch_types=[
            pltpu.VMEM((_CH, _GC), jnp.int32),
            pltpu.VMEM((_CROWS, EMB), jnp.float32),
            pltpu.SemaphoreType.DMA,
        ],
        compiler_params=pltpu.CompilerParams(use_tc_tiling_on_sc=False),
    )
    def k(idx_hbm, table_hbm, out_hbm, idx_v, rows_v, sem):
        wid = lax.axis_index("s") * 2 + lax.axis_index("c")

        def chunk(c, carry):
            blk = wid * _NCH + c
            pltpu.sync_copy(idx_hbm.at[blk], idx_v)
            for j in range(_CH):
                pltpu.async_copy(
                    table_hbm.at[idx_v.at[j]],
                    rows_v.at[pl.ds(j * _GC, _GC)],
                    sem,
                )
            for j in range(_CH):
                pltpu.make_async_copy(
                    table_hbm.at[idx_v.at[j]],
                    rows_v.at[pl.ds(j * _GC, _GC)],
                    sem,
                ).wait()
            pltpu.sync_copy(rows_v, out_hbm.at[pl.ds(blk * _CROWS, _CROWS)])
            return carry

        lax.fori_loop(0, _NCH, chunk, 0)

    return k(idx3d, table)


_KDIM = 104


def _sig(v):
    return 0.5 * jnp.tanh(0.5 * v) + 0.5


def _lstm_tc(emb_t, w_comb):
    def body(emb_ref, w_ref, out_ref, xh_ref, c_ref):
        t = pl.program_id(0)

        @pl.when(t == 0)
        def _init():
            row = lax.broadcasted_iota(jnp.int32, (_KDIM - EMB, BATCH), 0)
            xh_ref[EMB:_KDIM] = jnp.where(row == HID, 1.0, 0.0)
            c_ref[...] = jnp.zeros((HID, BATCH), jnp.float32)

        w = w_ref[...]
        x4t = emb_ref[...].reshape(BATCH // 4, 4 * EMB).T
        for j in range(4):
            q = pl.ds(j * (BATCH // 4), BATCH // 4)
            xh_ref[0:EMB, q] = x4t[j * EMB:(j + 1) * EMB]
            c = c_ref[:, q]
            gates = jnp.dot(w, xh_ref[:, q],
                            preferred_element_type=jnp.float32)
            i = _sig(gates[0:HID])
            f = _sig(gates[HID:2 * HID])
            g = jnp.tanh(gates[2 * HID:3 * HID])
            o = _sig(gates[3 * HID:4 * HID])
            c2 = f * c + i * g
            h2 = o * jnp.tanh(c2)
            xh_ref[EMB:EMB + HID, q] = h2
            c_ref[:, q] = c2

            @pl.when(t == SEQ - 1)
            def _fin():
                out_ref[:, q] = h2

    return pl.pallas_call(
        body,
        grid=(SEQ,),
        in_specs=[
            pl.BlockSpec((BATCH * EMB,), lambda i: (i,)),
            pl.BlockSpec((4 * HID, _KDIM), lambda i: (0, 0)),
        ],
        out_specs=pl.BlockSpec((HID, BATCH), lambda i: (0, 0)),
        out_shape=jax.ShapeDtypeStruct((HID, BATCH), jnp.float32),
        scratch_shapes=[
            pltpu.VMEM((_KDIM, BATCH), jnp.float32),
            pltpu.VMEM((HID, BATCH), jnp.float32),
        ],
    )(emb_t, w_comb)


def kernel(x, table, W_ih, W_hh, b_ih, b_hh):
    idxp = x.T.reshape(SEQ, 4, BATCH // 4).transpose(0, 2, 1)
    idx3d = idxp.reshape(_NW * _NCH, _CH, _GC)
    emb = _gather_sc(idx3d, table)
    emb_t = emb.reshape(SEQ * BATCH * EMB)
    w_comb = jnp.concatenate(
        [W_ih, W_hh, (b_ih + b_hh).reshape(4 * HID, 1),
         jnp.zeros((4 * HID, _KDIM - EMB - HID - 1), jnp.float32)],
        axis=1,
    )
    return _lstm_tc(emb_t, w_comb).T
